# baseline (speedup 1.0000x reference)
import functools

import jax
import jax.numpy as jnp
from jax import lax
from jax.experimental import pallas as pl
from jax.experimental.pallas import tpu as pltpu
from jax.experimental.pallas import tpu_sc as plsc

_NUM_CORES = 2
_NUM_SUBCORES = 16
_NW = _NUM_CORES * _NUM_SUBCORES
_CHUNK = 8
_NBUF = 6
_LOOKAHEAD = 3


def kernel(input_ids, embed_table):
    batch, seq = input_ids.shape
    _, d_model = embed_table.shape
    n = batch * seq
    b_per_w = n // _NW
    nchunks = b_per_w // _CHUNK
    L = _LOOKAHEAD
    assert nchunks >= 2 * _NBUF

    ids_flat = input_ids.reshape(n).astype(jnp.int32)

    mesh = plsc.VectorSubcoreMesh(
        core_axis_name="c", subcore_axis_name="s",
        num_cores=_NUM_CORES, num_subcores=_NUM_SUBCORES)

    scratch = [pltpu.VMEM((b_per_w,), jnp.int32)]
    scratch += [pltpu.VMEM((_CHUNK, d_model), jnp.float32)] * _NBUF
    scratch += [pltpu.SemaphoreType.DMA] * (2 * _NBUF)

    @functools.partial(
        pl.kernel,
        out_type=jax.ShapeDtypeStruct((n, d_model), jnp.float32),
        mesh=mesh,
        scratch_types=scratch,
    )
    def run(table_hbm, ids_hbm, out_hbm, idx_v, *rest):
        bufs = rest[:_NBUF]
        gsems = rest[_NBUF:2 * _NBUF]
        osems = rest[2 * _NBUF:]

        wid = lax.axis_index("s") * _NUM_CORES + lax.axis_index("c")
        base = wid * b_per_w
        pltpu.sync_copy(ids_hbm.at[pl.ds(base, b_per_w)], idx_v)

        def start_gather(i, b):
            pltpu.async_copy(
                table_hbm.at[idx_v.at[pl.ds(i * _CHUNK, _CHUNK)]],
                bufs[b], gsems[b])

        def wait_gather(i, b):
            pltpu.make_async_copy(
                table_hbm.at[idx_v.at[pl.ds(i * _CHUNK, _CHUNK)]],
                bufs[b], gsems[b]).wait()

        def start_out(i, b):
            pltpu.async_copy(
                bufs[b], out_hbm.at[pl.ds(base + i * _CHUNK, _CHUNK)],
                osems[b])

        def wait_out(i, b):
            pltpu.make_async_copy(
                bufs[b], out_hbm.at[pl.ds(base + i * _CHUNK, _CHUNK)],
                osems[b]).wait()

        for j in range(L):
            start_gather(j, j % _NBUF)

        def step(j, b):
            bL = (b + L) % _NBUF
            wait_out(j + L - _NBUF, bL)
            start_gather(j + L, bL)
            wait_gather(j, b)
            start_out(j, b)

        for j in range(_NBUF):
            b, bL = j % _NBUF, (j + L) % _NBUF
            if j + L - _NBUF >= 0:
                wait_out(j + L - _NBUF, bL)
            start_gather(j + L, bL)
            wait_gather(j, b)
            start_out(j, b)

        t1 = _NBUF + ((nchunks - L - 2 * _NBUF) // _NBUF) * _NBUF

        @pl.loop(_NBUF, t1, step=_NBUF)
        def _(g):
            for b in range(_NBUF):
                step(g + b, b)

        for j in range(t1, nchunks):
            b, bL = j % _NBUF, (j + L) % _NBUF
            if j + L < nchunks:
                wait_out(j + L - _NBUF, bL)
                start_gather(j + L, bL)
            wait_gather(j, b)
            start_out(j, b)
        for j in range(nchunks - _NBUF, nchunks):
            wait_out(j, j % _NBUF)

    out = run(embed_table, ids_flat)
    return out.reshape(batch, seq, d_model)


# speedup vs baseline: 1.0030x; 1.0030x over previous
import functools

import jax
import jax.numpy as jnp
from jax import lax
from jax.experimental import pallas as pl
from jax.experimental.pallas import tpu as pltpu
from jax.experimental.pallas import tpu_sc as plsc

_NUM_CORES = 2
_NUM_SUBCORES = 16
_NW = _NUM_CORES * _NUM_SUBCORES
_CHUNK = 8
_NBUF = 6
_LOOKAHEAD = 4


def kernel(input_ids, embed_table):
    batch, seq = input_ids.shape
    _, d_model = embed_table.shape
    n = batch * seq
    b_per_w = n // _NW
    nchunks = b_per_w // _CHUNK
    L = _LOOKAHEAD
    assert nchunks >= 2 * _NBUF

    ids_flat = input_ids.reshape(n).astype(jnp.int32)

    mesh = plsc.VectorSubcoreMesh(
        core_axis_name="c", subcore_axis_name="s",
        num_cores=_NUM_CORES, num_subcores=_NUM_SUBCORES)

    scratch = [pltpu.VMEM((b_per_w,), jnp.int32)]
    scratch += [pltpu.VMEM((_CHUNK, d_model), jnp.float32)] * _NBUF
    scratch += [pltpu.SemaphoreType.DMA] * (2 * _NBUF)

    @functools.partial(
        pl.kernel,
        out_type=jax.ShapeDtypeStruct((n, d_model), jnp.float32),
        mesh=mesh,
        scratch_types=scratch,
    )
    def run(table_hbm, ids_hbm, out_hbm, idx_v, *rest):
        bufs = rest[:_NBUF]
        gsems = rest[_NBUF:2 * _NBUF]
        osems = rest[2 * _NBUF:]

        wid = lax.axis_index("s") * _NUM_CORES + lax.axis_index("c")
        base = wid * b_per_w
        pltpu.sync_copy(ids_hbm.at[pl.ds(base, b_per_w)], idx_v)

        def start_gather(i, b):
            pltpu.async_copy(
                table_hbm.at[idx_v.at[pl.ds(i * _CHUNK, _CHUNK)]],
                bufs[b], gsems[b])

        def wait_gather(i, b):
            pltpu.make_async_copy(
                table_hbm.at[idx_v.at[pl.ds(i * _CHUNK, _CHUNK)]],
                bufs[b], gsems[b]).wait()

        def start_out(i, b):
            pltpu.async_copy(
                bufs[b], out_hbm.at[pl.ds(base + i * _CHUNK, _CHUNK)],
                osems[b])

        def wait_out(i, b):
            pltpu.make_async_copy(
                bufs[b], out_hbm.at[pl.ds(base + i * _CHUNK, _CHUNK)],
                osems[b]).wait()

        for j in range(L):
            start_gather(j, j % _NBUF)

        def step(j, b):
            bL = (b + L) % _NBUF
            wait_out(j + L - _NBUF, bL)
            start_gather(j + L, bL)
            wait_gather(j, b)
            start_out(j, b)

        for j in range(_NBUF):
            b, bL = j % _NBUF, (j + L) % _NBUF
            if j + L - _NBUF >= 0:
                wait_out(j + L - _NBUF, bL)
            start_gather(j + L, bL)
            wait_gather(j, b)
            start_out(j, b)

        t1 = _NBUF + ((nchunks - L - 2 * _NBUF) // _NBUF) * _NBUF

        @pl.loop(_NBUF, t1, step=_NBUF)
        def _(g):
            for b in range(_NBUF):
                step(g + b, b)

        for j in range(t1, nchunks):
            b, bL = j % _NBUF, (j + L) % _NBUF
            if j + L < nchunks:
                wait_out(j + L - _NBUF, bL)
                start_gather(j + L, bL)
            wait_gather(j, b)
            start_out(j, b)
        for j in range(nchunks - _NBUF, nchunks):
            wait_out(j, j % _NBUF)

    out = run(embed_table, ids_flat)
    return out.reshape(batch, seq, d_model)
